# baseline (speedup 1.0000x reference)
import functools

import jax
import jax.numpy as jnp
from jax import lax
from jax.experimental import pallas as pl
from jax.experimental.pallas import tpu as pltpu
from jax.experimental.pallas import tpu_sc as plsc

BATCH = 16384
EMB = 32
L = 16

_info = plsc.get_sparse_core_info()
NC = _info.num_cores
NS = _info.num_subcores
NW = NC * NS
BPW = BATCH // NW

TC_BLK = 32768


def _col_dots(user_t, item_t, w_row):
    nu = user_t.shape[1]
    ni = item_t.shape[1]

    def body(u_ref, i_ref, w_ref, su_ref, si_ref):
        w = w_ref[...]
        su_ref[...] = jax.lax.dot_general(
            w[0:1, 0:EMB], u_ref[...], (((1,), (0,)), ((), ())),
            preferred_element_type=jnp.float32).reshape(-1)

        @pl.when(pl.program_id(0) == 0)
        def _():
            si_ref[...] = jax.lax.dot_general(
                w[0:1, EMB:2 * EMB], i_ref[...], (((1,), (0,)), ((), ())),
                preferred_element_type=jnp.float32).reshape(-1)

    return pl.pallas_call(
        body,
        grid=(pl.cdiv(nu, TC_BLK),),
        in_specs=[
            pl.BlockSpec((EMB, TC_BLK), lambda i: (0, i)),
            pl.BlockSpec((EMB, ni), lambda i: (0, 0)),
            pl.BlockSpec((1, 2 * EMB), lambda i: (0, 0)),
        ],
        out_specs=[
            pl.BlockSpec((TC_BLK,), lambda i: (i,)),
            pl.BlockSpec((ni,), lambda i: (0,)),
        ],
        out_shape=[
            jax.ShapeDtypeStruct((nu,), jnp.float32),
            jax.ShapeDtypeStruct((ni,), jnp.float32),
        ],
    )(user_t, item_t, w_row)


def _sc_body(uid_hbm, iid_hbm, su_hbm, si_hbm, wb_hbm, out_hbm,
             idx_u, idx_i, g_u, g_i, out_v, wv, sem_u, sem_i):
    wid = lax.axis_index("s") * NC + lax.axis_index("c")
    base = wid * BPW

    pltpu.sync_copy(wb_hbm, wv)
    pltpu.sync_copy(uid_hbm.at[pl.ds(base, BPW)], idx_u)
    pltpu.sync_copy(iid_hbm.at[pl.ds(base, BPW)], idx_i)

    cu = pltpu.async_copy(su_hbm.at[idx_u], g_u, sem_u)
    ci = pltpu.async_copy(si_hbm.at[idx_i], g_i, sem_i)
    cu.wait()
    ci.wait()

    bias_vec = wv[pl.ds(0, L)]

    def group(g, carry):
        r0 = g * L
        out_v[pl.ds(r0, L)] = (
            g_u[pl.ds(r0, L)] + g_i[pl.ds(r0, L)] + bias_vec)
        return carry

    lax.fori_loop(0, BPW // L, group, 0)
    pltpu.sync_copy(out_v, out_hbm.at[pl.ds(base, BPW)])


@jax.jit
def _run(user_ids, item_ids, user_table, item_table, fc_w, fc_b):
    su, si = _col_dots(user_table.T, item_table.T, fc_w)
    wb = jnp.broadcast_to(fc_b.reshape(-1), (L,))

    mesh = plsc.VectorSubcoreMesh(core_axis_name="c", subcore_axis_name="s")
    k = functools.partial(
        pl.kernel,
        mesh=mesh,
        compiler_params=pltpu.CompilerParams(
            needs_layout_passes=False, use_tc_tiling_on_sc=False),
        out_type=jax.ShapeDtypeStruct((BATCH,), jnp.float32),
        scratch_types=[
            pltpu.VMEM((BPW,), jnp.int32),
            pltpu.VMEM((BPW,), jnp.int32),
            pltpu.VMEM((BPW,), jnp.float32),
            pltpu.VMEM((BPW,), jnp.float32),
            pltpu.VMEM((BPW,), jnp.float32),
            pltpu.VMEM((L,), jnp.float32),
            pltpu.SemaphoreType.DMA,
            pltpu.SemaphoreType.DMA,
        ],
    )(_sc_body)
    return k(user_ids, item_ids, su, si, wb)


def kernel(user_ids, item_ids, user_table, item_table, fc_w, fc_b):
    return _run(user_ids.astype(jnp.int32), item_ids.astype(jnp.int32),
                user_table, item_table, fc_w, fc_b)


# speedup vs baseline: 1.0704x; 1.0704x over previous
import functools

import jax
import jax.numpy as jnp
from jax import lax
from jax.experimental import pallas as pl
from jax.experimental.pallas import tpu as pltpu
from jax.experimental.pallas import tpu_sc as plsc

BATCH = 16384
EMB = 32
L = 16

_info = plsc.get_sparse_core_info()
NC = _info.num_cores
NS = _info.num_subcores
NW = NC * NS
BPW = BATCH // NW

TC_BLK = 98304


def _col_dots(user_t, item_t, w_row):
    nu = user_t.shape[1]
    ni = item_t.shape[1]

    def body(u_ref, i_ref, w_ref, su_ref, si_ref):
        w = w_ref[...]
        su_ref[...] = jax.lax.dot_general(
            w[0:1, 0:EMB], u_ref[...], (((1,), (0,)), ((), ())),
            preferred_element_type=jnp.float32).reshape(-1)

        @pl.when(pl.program_id(0) == 0)
        def _():
            si_ref[...] = jax.lax.dot_general(
                w[0:1, EMB:2 * EMB], i_ref[...], (((1,), (0,)), ((), ())),
                preferred_element_type=jnp.float32).reshape(-1)

    return pl.pallas_call(
        body,
        grid=(pl.cdiv(nu, TC_BLK),),
        in_specs=[
            pl.BlockSpec((EMB, TC_BLK), lambda i: (0, i)),
            pl.BlockSpec((EMB, ni), lambda i: (0, 0)),
            pl.BlockSpec((1, 2 * EMB), lambda i: (0, 0)),
        ],
        out_specs=[
            pl.BlockSpec((TC_BLK,), lambda i: (i,)),
            pl.BlockSpec((ni,), lambda i: (0,)),
        ],
        out_shape=[
            jax.ShapeDtypeStruct((nu,), jnp.float32),
            jax.ShapeDtypeStruct((ni,), jnp.float32),
        ],
    )(user_t, item_t, w_row)


def _sc_body(uid_hbm, iid_hbm, su_hbm, si_hbm, wb_hbm, out_hbm,
             idx_u, idx_i, g_u, g_i, out_v, wv, sem_u, sem_i):
    wid = lax.axis_index("s") * NC + lax.axis_index("c")
    base = wid * BPW

    pltpu.sync_copy(wb_hbm, wv)
    pltpu.sync_copy(uid_hbm.at[pl.ds(base, BPW)], idx_u)
    pltpu.sync_copy(iid_hbm.at[pl.ds(base, BPW)], idx_i)

    cu = pltpu.async_copy(su_hbm.at[idx_u], g_u, sem_u)
    ci = pltpu.async_copy(si_hbm.at[idx_i], g_i, sem_i)
    cu.wait()
    ci.wait()

    bias_vec = wv[pl.ds(0, L)]

    def group(g, carry):
        r0 = g * L
        out_v[pl.ds(r0, L)] = (
            g_u[pl.ds(r0, L)] + g_i[pl.ds(r0, L)] + bias_vec)
        return carry

    lax.fori_loop(0, BPW // L, group, 0)
    pltpu.sync_copy(out_v, out_hbm.at[pl.ds(base, BPW)])


@jax.jit
def _run(user_ids, item_ids, user_table, item_table, fc_w, fc_b):
    su, si = _col_dots(user_table.T, item_table.T, fc_w)
    wb = jnp.broadcast_to(fc_b.reshape(-1), (L,))

    mesh = plsc.VectorSubcoreMesh(core_axis_name="c", subcore_axis_name="s")
    k = functools.partial(
        pl.kernel,
        mesh=mesh,
        compiler_params=pltpu.CompilerParams(
            needs_layout_passes=False, use_tc_tiling_on_sc=False),
        out_type=jax.ShapeDtypeStruct((BATCH,), jnp.float32),
        scratch_types=[
            pltpu.VMEM((BPW,), jnp.int32),
            pltpu.VMEM((BPW,), jnp.int32),
            pltpu.VMEM((BPW,), jnp.float32),
            pltpu.VMEM((BPW,), jnp.float32),
            pltpu.VMEM((BPW,), jnp.float32),
            pltpu.VMEM((L,), jnp.float32),
            pltpu.SemaphoreType.DMA,
            pltpu.SemaphoreType.DMA,
        ],
    )(_sc_body)
    return k(user_ids, item_ids, su, si, wb)


def kernel(user_ids, item_ids, user_table, item_table, fc_w, fc_b):
    return _run(user_ids.astype(jnp.int32), item_ids.astype(jnp.int32),
                user_table, item_table, fc_w, fc_b)


# speedup vs baseline: 1.0828x; 1.0116x over previous
import functools

import jax
import jax.numpy as jnp
from jax import lax
from jax.experimental import pallas as pl
from jax.experimental.pallas import tpu as pltpu
from jax.experimental.pallas import tpu_sc as plsc

BATCH = 16384
EMB = 32
L = 16

_info = plsc.get_sparse_core_info()
NC = _info.num_cores
NS = _info.num_subcores
NW = NC * NS
BPW = BATCH // NW

TC_BLK = 65536


def _col_dots(user_t, item_t, w_row):
    nu = user_t.shape[1]
    ni = item_t.shape[1]

    def body(u_ref, i_ref, w_ref, su_ref, si_ref):
        w = w_ref[...]
        su_ref[...] = jax.lax.dot_general(
            w[0:1, 0:EMB], u_ref[...], (((1,), (0,)), ((), ())),
            preferred_element_type=jnp.float32).reshape(-1)

        @pl.when(pl.program_id(0) == 0)
        def _():
            si_ref[...] = jax.lax.dot_general(
                w[0:1, EMB:2 * EMB], i_ref[...], (((1,), (0,)), ((), ())),
                preferred_element_type=jnp.float32).reshape(-1)

    return pl.pallas_call(
        body,
        grid=(pl.cdiv(nu, TC_BLK),),
        in_specs=[
            pl.BlockSpec((EMB, TC_BLK), lambda i: (0, i)),
            pl.BlockSpec((EMB, ni), lambda i: (0, 0)),
            pl.BlockSpec((1, 2 * EMB), lambda i: (0, 0)),
        ],
        out_specs=[
            pl.BlockSpec((TC_BLK,), lambda i: (i,)),
            pl.BlockSpec((ni,), lambda i: (0,)),
        ],
        out_shape=[
            jax.ShapeDtypeStruct((nu,), jnp.float32),
            jax.ShapeDtypeStruct((ni,), jnp.float32),
        ],
    )(user_t, item_t, w_row)


def _sc_body(uid_hbm, iid_hbm, su_hbm, si_hbm, wb_hbm, out_hbm,
             idx_u, idx_i, g_u, g_i, out_v, wv, sem_u, sem_i):
    wid = lax.axis_index("s") * NC + lax.axis_index("c")
    base = wid * BPW

    pltpu.sync_copy(wb_hbm, wv)
    pltpu.sync_copy(uid_hbm.at[pl.ds(base, BPW)], idx_u)
    pltpu.sync_copy(iid_hbm.at[pl.ds(base, BPW)], idx_i)

    cu = pltpu.async_copy(su_hbm.at[idx_u], g_u, sem_u)
    ci = pltpu.async_copy(si_hbm.at[idx_i], g_i, sem_i)
    cu.wait()
    ci.wait()

    bias_vec = wv[pl.ds(0, L)]

    def group(g, carry):
        r0 = g * L
        out_v[pl.ds(r0, L)] = (
            g_u[pl.ds(r0, L)] + g_i[pl.ds(r0, L)] + bias_vec)
        return carry

    lax.fori_loop(0, BPW // L, group, 0)
    pltpu.sync_copy(out_v, out_hbm.at[pl.ds(base, BPW)])


@jax.jit
def _run(user_ids, item_ids, user_table, item_table, fc_w, fc_b):
    su, si = _col_dots(user_table.T, item_table.T, fc_w)
    wb = jnp.broadcast_to(fc_b.reshape(-1), (L,))

    mesh = plsc.VectorSubcoreMesh(core_axis_name="c", subcore_axis_name="s")
    k = functools.partial(
        pl.kernel,
        mesh=mesh,
        compiler_params=pltpu.CompilerParams(
            needs_layout_passes=False, use_tc_tiling_on_sc=False),
        out_type=jax.ShapeDtypeStruct((BATCH,), jnp.float32),
        scratch_types=[
            pltpu.VMEM((BPW,), jnp.int32),
            pltpu.VMEM((BPW,), jnp.int32),
            pltpu.VMEM((BPW,), jnp.float32),
            pltpu.VMEM((BPW,), jnp.float32),
            pltpu.VMEM((BPW,), jnp.float32),
            pltpu.VMEM((L,), jnp.float32),
            pltpu.SemaphoreType.DMA,
            pltpu.SemaphoreType.DMA,
        ],
    )(_sc_body)
    return k(user_ids, item_ids, su, si, wb)


def kernel(user_ids, item_ids, user_table, item_table, fc_w, fc_b):
    return _run(user_ids.astype(jnp.int32), item_ids.astype(jnp.int32),
                user_table, item_table, fc_w, fc_b)


# speedup vs baseline: 1.0997x; 1.0156x over previous
import functools

import jax
import jax.numpy as jnp
from jax import lax
from jax.experimental import pallas as pl
from jax.experimental.pallas import tpu as pltpu
from jax.experimental.pallas import tpu_sc as plsc

BATCH = 16384
EMB = 32
L = 16

_info = plsc.get_sparse_core_info()
NC = _info.num_cores
NS = _info.num_subcores
NW = NC * NS
BPW = BATCH // NW

TC_BLK = 65536


def _col_dots(user_t, item_t, w_row, bias):
    nu = user_t.shape[1]
    ni = item_t.shape[1]

    def body(u_ref, i_ref, w_ref, b_ref, su_ref, si_ref):
        w = w_ref[...]
        su_ref[...] = jax.lax.dot_general(
            w[0:1, 0:EMB], u_ref[...], (((1,), (0,)), ((), ())),
            preferred_element_type=jnp.float32).reshape(-1)

        @pl.when(pl.program_id(0) == 0)
        def _():
            si_ref[...] = jax.lax.dot_general(
                w[0:1, EMB:2 * EMB], i_ref[...], (((1,), (0,)), ((), ())),
                preferred_element_type=jnp.float32).reshape(-1) + b_ref[...]

    return pl.pallas_call(
        body,
        grid=(pl.cdiv(nu, TC_BLK),),
        in_specs=[
            pl.BlockSpec((EMB, TC_BLK), lambda i: (0, i)),
            pl.BlockSpec((EMB, ni), lambda i: (0, 0)),
            pl.BlockSpec((1, 2 * EMB), lambda i: (0, 0)),
            pl.BlockSpec((1,), lambda i: (0,)),
        ],
        out_specs=[
            pl.BlockSpec((TC_BLK,), lambda i: (i,)),
            pl.BlockSpec((ni,), lambda i: (0,)),
        ],
        out_shape=[
            jax.ShapeDtypeStruct((nu,), jnp.float32),
            jax.ShapeDtypeStruct((ni,), jnp.float32),
        ],
    )(user_t, item_t, w_row, bias)


def _sc_body(uid_hbm, iid_hbm, su_hbm, si_hbm, out_hbm,
             idx_u, idx_i, g_u, g_i, out_v, sem_u, sem_i):
    wid = lax.axis_index("s") * NC + lax.axis_index("c")
    base = wid * BPW

    pltpu.sync_copy(uid_hbm.at[pl.ds(base, BPW)], idx_u)
    pltpu.sync_copy(iid_hbm.at[pl.ds(base, BPW)], idx_i)

    cu = pltpu.async_copy(su_hbm.at[idx_u], g_u, sem_u)
    ci = pltpu.async_copy(si_hbm.at[idx_i], g_i, sem_i)
    cu.wait()
    ci.wait()

    def group(g, carry):
        r0 = g * L
        out_v[pl.ds(r0, L)] = g_u[pl.ds(r0, L)] + g_i[pl.ds(r0, L)]
        return carry

    lax.fori_loop(0, BPW // L, group, 0)
    pltpu.sync_copy(out_v, out_hbm.at[pl.ds(base, BPW)])


@jax.jit
def _run(user_ids, item_ids, user_table, item_table, fc_w, fc_b):
    su, si = _col_dots(user_table.T, item_table.T, fc_w, fc_b)

    mesh = plsc.VectorSubcoreMesh(core_axis_name="c", subcore_axis_name="s")
    k = functools.partial(
        pl.kernel,
        mesh=mesh,
        compiler_params=pltpu.CompilerParams(
            needs_layout_passes=False, use_tc_tiling_on_sc=False),
        out_type=jax.ShapeDtypeStruct((BATCH,), jnp.float32),
        scratch_types=[
            pltpu.VMEM((BPW,), jnp.int32),
            pltpu.VMEM((BPW,), jnp.int32),
            pltpu.VMEM((BPW,), jnp.float32),
            pltpu.VMEM((BPW,), jnp.float32),
            pltpu.VMEM((BPW,), jnp.float32),
            pltpu.SemaphoreType.DMA,
            pltpu.SemaphoreType.DMA,
        ],
    )(_sc_body)
    return k(user_ids, item_ids, su, si)


def kernel(user_ids, item_ids, user_table, item_table, fc_w, fc_b):
    return _run(user_ids.astype(jnp.int32), item_ids.astype(jnp.int32),
                user_table, item_table, fc_w, fc_b)


# speedup vs baseline: 1.1067x; 1.0064x over previous
import functools

import jax
import jax.numpy as jnp
from jax import lax
from jax.experimental import pallas as pl
from jax.experimental.pallas import tpu as pltpu
from jax.experimental.pallas import tpu_sc as plsc

BATCH = 16384
EMB = 32
L = 16

_info = plsc.get_sparse_core_info()
NC = _info.num_cores
NS = _info.num_subcores
NW = NC * NS
BPW = BATCH // NW

TC_BLK = 65536


def _col_dots(user_t, item_t, w_row, bias):
    nu = user_t.shape[1]
    ni = item_t.shape[1]

    def body(u_ref, i_ref, w_ref, b_ref, su_ref, si_ref):
        w = w_ref[...]
        su_ref[...] = jax.lax.dot_general(
            w[0:1, 0:EMB], u_ref[...], (((1,), (0,)), ((), ())),
            preferred_element_type=jnp.float32).reshape(-1)

        @pl.when(pl.program_id(0) == 0)
        def _():
            si_ref[...] = jax.lax.dot_general(
                w[0:1, EMB:2 * EMB], i_ref[...], (((1,), (0,)), ((), ())),
                preferred_element_type=jnp.float32).reshape(-1) + b_ref[...]

    return pl.pallas_call(
        body,
        grid=(pl.cdiv(nu, TC_BLK),),
        in_specs=[
            pl.BlockSpec((EMB, TC_BLK), lambda i: (0, i)),
            pl.BlockSpec((EMB, ni), lambda i: (0, 0)),
            pl.BlockSpec((1, 2 * EMB), lambda i: (0, 0)),
            pl.BlockSpec((1,), lambda i: (0,)),
        ],
        out_specs=[
            pl.BlockSpec((TC_BLK,), lambda i: (i,)),
            pl.BlockSpec((ni,), lambda i: (0,)),
        ],
        out_shape=[
            jax.ShapeDtypeStruct((nu,), jnp.float32),
            jax.ShapeDtypeStruct((ni,), jnp.float32),
        ],
    )(user_t, item_t, w_row, bias)


def _sc_body(uid_hbm, iid_hbm, su_hbm, si_hbm, out_hbm,
             idx_u, idx_i, g_u, g_i, out_v, sem_u, sem_i):
    wid = lax.axis_index("s") * NC + lax.axis_index("c")
    base = wid * BPW

    ku = pltpu.async_copy(uid_hbm.at[pl.ds(base, BPW)], idx_u, sem_u)
    ki = pltpu.async_copy(iid_hbm.at[pl.ds(base, BPW)], idx_i, sem_i)
    ku.wait()
    cu = pltpu.async_copy(su_hbm.at[idx_u], g_u, sem_u)
    ki.wait()
    ci = pltpu.async_copy(si_hbm.at[idx_i], g_i, sem_i)
    cu.wait()
    ci.wait()

    def group(g, carry):
        r0 = g * L
        out_v[pl.ds(r0, L)] = g_u[pl.ds(r0, L)] + g_i[pl.ds(r0, L)]
        return carry

    lax.fori_loop(0, BPW // L, group, 0)
    pltpu.sync_copy(out_v, out_hbm.at[pl.ds(base, BPW)])


@jax.jit
def _run(user_ids, item_ids, user_table, item_table, fc_w, fc_b):
    su, si = _col_dots(user_table.T, item_table.T, fc_w, fc_b)

    mesh = plsc.VectorSubcoreMesh(core_axis_name="c", subcore_axis_name="s")
    k = functools.partial(
        pl.kernel,
        mesh=mesh,
        compiler_params=pltpu.CompilerParams(
            needs_layout_passes=False, use_tc_tiling_on_sc=False),
        out_type=jax.ShapeDtypeStruct((BATCH,), jnp.float32),
        scratch_types=[
            pltpu.VMEM((BPW,), jnp.int32),
            pltpu.VMEM((BPW,), jnp.int32),
            pltpu.VMEM((BPW,), jnp.float32),
            pltpu.VMEM((BPW,), jnp.float32),
            pltpu.VMEM((BPW,), jnp.float32),
            pltpu.SemaphoreType.DMA,
            pltpu.SemaphoreType.DMA,
        ],
    )(_sc_body)
    return k(user_ids, item_ids, su, si)


def kernel(user_ids, item_ids, user_table, item_table, fc_w, fc_b):
    return _run(user_ids.astype(jnp.int32), item_ids.astype(jnp.int32),
                user_table, item_table, fc_w, fc_b)
